# baseline (speedup 1.0000x reference)
import functools

import jax
import jax.numpy as jnp
from jax import lax
from jax.experimental import pallas as pl
from jax.experimental.pallas import tpu as pltpu
from jax.experimental.pallas import tpu_sc as plsc

_N = 10000
_E = 320000
_D = 128
_G = 64
_NSUB = 16
_NCORE = 2
_NW = _NSUB * _NCORE
_K = 128
_NCH = 80
_HCH = 40
_EPAD = _NW * _NCH * _K
_ROWS_A = 640
_ROWS_B = _N - 15 * _ROWS_A
_NPAD = _N + 16
_BLK = 2000
_NBLK = _N // _BLK


def _sc_agg_body(x_hbm, idx_hbm, out_hbm,
                 pk_v, src_v, dst_v, rows0, rows1, acc, gsem0, gsem1):
    c = lax.axis_index("c")
    s = lax.axis_index("s")
    wid = c * _NSUB + s

    off = pl.multiple_of(s * _ROWS_A, 8)

    @pl.when(s < _NSUB - 1)
    def _():
        pltpu.sync_copy(x_hbm.at[pl.ds(off, _ROWS_A)],
                        acc.at[pl.ds(off, _ROWS_A)])

    @pl.when(s == _NSUB - 1)
    def _():
        pltpu.sync_copy(x_hbm.at[pl.ds(15 * _ROWS_A, _ROWS_B)],
                        acc.at[pl.ds(15 * _ROWS_A, _ROWS_B)])

    plsc.subcore_barrier()

    for half in range(2):
        pltpu.sync_copy(
            idx_hbm.at[pl.ds(pl.multiple_of(wid * _NCH + half * _HCH, 8),
                             _HCH)], pk_v)

        def unpack(j, carry):
            for k in range(_K // 16):
                pv = pk_v[j, pl.ds(k * 16, 16)]
                src_v[j, pl.ds(k * 16, 16)] = lax.bitwise_and(pv, 0x3FFF)
                dst_v[j, pl.ds(k * 16, 16)] = lax.shift_right_logical(pv, 14)
            return carry

        lax.fori_loop(0, _HCH, unpack, 0)

        def gather(j, rows, gsem):
            pltpu.async_copy(x_hbm.at[src_v.at[j, pl.ds(0, 64)]],
                             rows.at[pl.ds(0, 64)], gsem)
            pltpu.async_copy(x_hbm.at[src_v.at[j, pl.ds(64, 64)]],
                             rows.at[pl.ds(64, 64)], gsem)

        gather(0, rows0, gsem0)
        gather(1, rows1, gsem1)

        def step(g, carry):
            for b, (rows, gsem) in enumerate(((rows0, gsem0),
                                              (rows1, gsem1))):
                j = g * 2 + b
                pltpu.make_async_copy(x_hbm.at[src_v.at[j]], rows,
                                      gsem).wait()
                pltpu.sync_copy(rows, acc.at[dst_v.at[j]], add=True)
                nj = j + 2

                @pl.when(nj < _HCH)
                def _():
                    gather(nj, rows, gsem)
            return carry

        lax.fori_loop(0, _HCH // 2, step, 0)

    plsc.subcore_barrier()

    @pl.when(s < _NSUB - 1)
    def _():
        pltpu.sync_copy(acc.at[pl.ds(off, _ROWS_A)],
                        out_hbm.at[c, pl.ds(off, _ROWS_A)])

    @pl.when(s == _NSUB - 1)
    def _():
        pltpu.sync_copy(acc.at[pl.ds(15 * _ROWS_A, _ROWS_B)],
                        out_hbm.at[c, pl.ds(15 * _ROWS_A, _ROWS_B)])


@functools.lru_cache(maxsize=None)
def _get_sc_agg():
    return pl.kernel(
        _sc_agg_body,
        out_type=jax.ShapeDtypeStruct((_NCORE, _N, _D), jnp.float32),
        mesh=plsc.VectorSubcoreMesh(core_axis_name="c", subcore_axis_name="s"),
        scratch_types=[
            pltpu.VMEM((_HCH, _K), jnp.int32),
            pltpu.VMEM((_HCH, _K), jnp.int32),
            pltpu.VMEM((_HCH, _K), jnp.int32),
            pltpu.VMEM((_K, _D), jnp.float32),
            pltpu.VMEM((_K, _D), jnp.float32),
            pltpu.VMEM_SHARED((_NPAD, _D), jnp.float32),
            pltpu.SemaphoreType.DMA,
            pltpu.SemaphoreType.DMA,
        ],
    )


def _sc_agg(h, idx3):
    return _get_sc_agg()(h, idx3)


def _k1_body(x_ref, p_ref, w1_ref, b1_ref, w2_ref, b2_ref, h2_ref, st_ref):
    i = pl.program_id(0)
    t = p_ref[0] + p_ref[1] - x_ref[...]
    h1 = jnp.maximum(
        jnp.dot(t, w1_ref[...], preferred_element_type=jnp.float32)
        + b1_ref[...], 0.0)
    h2 = jnp.maximum(
        jnp.dot(h1, w2_ref[...], preferred_element_type=jnp.float32)
        + b2_ref[...], 0.0)
    h2_ref[...] = h2

    @pl.when(i == 0)
    def _():
        st_ref[...] = jnp.zeros_like(st_ref)

    upd = jnp.concatenate(
        [jnp.sum(h2, axis=0, keepdims=True),
         jnp.sum(h2 * h2, axis=0, keepdims=True)], axis=0)
    st_ref[0:2, :] = st_ref[0:2, :] + upd


def _k1(x, p, w1, b1, w2, b2):
    return pl.pallas_call(
        _k1_body,
        grid=(_NBLK,),
        in_specs=[
            pl.BlockSpec((_BLK, _D), lambda i: (i, 0)),
            pl.BlockSpec((2, _BLK, _D), lambda i: (0, i, 0)),
            pl.BlockSpec((_D, _D), lambda i: (0, 0)),
            pl.BlockSpec((1, _D), lambda i: (0, 0)),
            pl.BlockSpec((_D, _D), lambda i: (0, 0)),
            pl.BlockSpec((1, _D), lambda i: (0, 0)),
        ],
        out_specs=[
            pl.BlockSpec((_BLK, _D), lambda i: (i, 0)),
            pl.BlockSpec((8, _D), lambda i: (0, 0)),
        ],
        out_shape=[
            jax.ShapeDtypeStruct((_N, _D), jnp.float32),
            jax.ShapeDtypeStruct((8, _D), jnp.float32),
        ],
    )(x, p, w1, b1.reshape(1, _D), w2, b2.reshape(1, _D))


def _norm(h2, st_ref, g_ref, be_ref):
    mean = st_ref[0:1, :] * (1.0 / _N)
    var = st_ref[1:2, :] * (1.0 / _N) - mean * mean
    rstd = lax.rsqrt(var + 1e-5)
    return (h2 - mean) * (rstd * g_ref[...]) + be_ref[...]


def _k2_body(h2_ref, st_ref, g_ref, be_ref, out_ref):
    out_ref[...] = _norm(h2_ref[...], st_ref, g_ref, be_ref)


def _k2(h2, st, g, be):
    return pl.pallas_call(
        _k2_body,
        grid=(_NBLK,),
        in_specs=[
            pl.BlockSpec((_BLK, _D), lambda i: (i, 0)),
            pl.BlockSpec((8, _D), lambda i: (0, 0)),
            pl.BlockSpec((1, _D), lambda i: (0, 0)),
            pl.BlockSpec((1, _D), lambda i: (0, 0)),
        ],
        out_specs=pl.BlockSpec((_BLK, _D), lambda i: (i, 0)),
        out_shape=jax.ShapeDtypeStruct((_N, _D), jnp.float32),
    )(h2, st, g.reshape(1, _D), be.reshape(1, _D))


def _k3_body(h2_ref, st_ref, g_ref, be_ref, b_ref, pooled_ref, sums, cnt):
    i = pl.program_id(0)
    h = _norm(h2_ref[...], st_ref, g_ref, be_ref)
    lbl = b_ref[0]
    ids = lax.broadcasted_iota(jnp.int32, (_G, _BLK), 0)
    oh = (lbl == ids).astype(jnp.float32)

    @pl.when(i == 0)
    def _():
        sums[...] = jnp.zeros_like(sums)
        cnt[...] = jnp.zeros_like(cnt)

    sums[...] = sums[...] + jnp.dot(oh, h, preferred_element_type=jnp.float32)
    cnt[...] = cnt[...] + jnp.broadcast_to(
        jnp.sum(oh, axis=1, keepdims=True), (_G, _D))

    @pl.when(i == _NBLK - 1)
    def _():
        pooled_ref[...] = sums[...] / jnp.maximum(cnt[...], 1.0)


def _k3(h2, st, g, be, batch3):
    return pl.pallas_call(
        _k3_body,
        grid=(_NBLK,),
        in_specs=[
            pl.BlockSpec((_BLK, _D), lambda i: (i, 0)),
            pl.BlockSpec((8, _D), lambda i: (0, 0)),
            pl.BlockSpec((1, _D), lambda i: (0, 0)),
            pl.BlockSpec((1, _D), lambda i: (0, 0)),
            pl.BlockSpec((1, 1, _BLK), lambda i: (i, 0, 0)),
        ],
        out_specs=pl.BlockSpec((_G, _D), lambda i: (0, 0)),
        out_shape=jax.ShapeDtypeStruct((_G, _D), jnp.float32),
        scratch_shapes=[
            pltpu.VMEM((_G, _D), jnp.float32),
            pltpu.VMEM((_G, _D), jnp.float32),
        ],
    )(h2, st, g.reshape(1, _D), be.reshape(1, _D), batch3)


def _k4_body(p_ref, fw_ref, fb_ref, sw_ref, sb_ref, out_ref):
    h = jnp.maximum(
        jnp.dot(p_ref[...], fw_ref[...], preferred_element_type=jnp.float32)
        + fb_ref[...], 0.0)
    logits = jnp.dot(h, sw_ref[...],
                     preferred_element_type=jnp.float32) + sb_ref[...]
    m = jnp.max(logits, axis=-1, keepdims=True)
    lse = jnp.log(jnp.sum(jnp.exp(logits - m), axis=-1, keepdims=True)) + m
    out_ref[...] = logits - lse


def _k4(pooled, fw, fb, sw, sb):
    return pl.pallas_call(
        _k4_body,
        out_shape=jax.ShapeDtypeStruct((_G, 16), jnp.float32),
    )(pooled, fw, fb.reshape(1, _D), sw, sb.reshape(1, 16))


def kernel(x, edge_index, batch,
           W1_0, b1_0, W2_0, b2_0, g_0, be_0,
           W1_1, b1_1, W2_1, b2_1, g_1, be_1,
           W1_2, b1_2, W2_2, b2_2, g_2, be_2,
           first_W, first_b, second_W, second_b):
    padlen = _EPAD - _E
    fill = jnp.arange(padlen, dtype=jnp.int32)
    src3 = jnp.concatenate(
        [edge_index[0], (fill * 97) % _N]).reshape(_NW, _NCH, _K)
    dst3 = jnp.concatenate(
        [edge_index[1], _N + (fill % 16)]).reshape(_NW, _NCH, _K)
    idx3 = jnp.pad(((dst3 << 14) | src3).reshape(_NW * _NCH, _K),
                   ((0, 16384 - _NW * _NCH), (0, 0)))
    batch3 = batch.reshape(_NBLK, 1, _BLK)

    h = x
    params = ((W1_0, b1_0, W2_0, b2_0, g_0, be_0),
              (W1_1, b1_1, W2_1, b2_1, g_1, be_1),
              (W1_2, b1_2, W2_2, b2_2, g_2, be_2))
    for li, (w1, b1, w2, b2, g, be) in enumerate(params):
        p = _sc_agg(h, idx3)
        h2, st = _k1(h, p, w1, b1, w2, b2)
        if li < 2:
            h = _k2(h2, st, g, be)
        else:
            pooled = _k3(h2, st, g, be, batch3)
    return _k4(pooled, first_W, first_b, second_W, second_b)


# speedup vs baseline: 1.0078x; 1.0078x over previous
import functools

import jax
import jax.numpy as jnp
from jax import lax
from jax.experimental import pallas as pl
from jax.experimental.pallas import tpu as pltpu
from jax.experimental.pallas import tpu_sc as plsc

_N = 10000
_E = 320000
_D = 128
_G = 64
_NSUB = 16
_NCORE = 2
_NW = _NSUB * _NCORE
_K = 128
_NCH = 80
_HCH = 40
_EPAD = _NW * _NCH * _K
_ROWS_A = 640
_ROWS_B = _N - 15 * _ROWS_A
_NPAD = _N + 16
_BLK = 2000
_NBLK = _N // _BLK


def _sc_agg_body(x_hbm, idx_hbm, out_hbm,
                 pk_v, src_v, dst_v, rows0, rows1, acc, gsem0, gsem1):
    c = lax.axis_index("c")
    s = lax.axis_index("s")
    wid = c * _NSUB + s

    off = pl.multiple_of(s * _ROWS_A, 8)

    @pl.when(s < _NSUB - 1)
    def _():
        pltpu.sync_copy(x_hbm.at[pl.ds(off, _ROWS_A)],
                        acc.at[pl.ds(off, _ROWS_A)])

    @pl.when(s == _NSUB - 1)
    def _():
        pltpu.sync_copy(x_hbm.at[pl.ds(15 * _ROWS_A, _ROWS_B)],
                        acc.at[pl.ds(15 * _ROWS_A, _ROWS_B)])

    plsc.subcore_barrier()

    for half in range(2):
        pltpu.sync_copy(
            idx_hbm.at[pl.ds(pl.multiple_of(wid * _NCH + half * _HCH, 8),
                             _HCH)], pk_v)

        def unpack(j, carry):
            for k in range(_K // 16):
                pv = pk_v[j, pl.ds(k * 16, 16)]
                src_v[j, pl.ds(k * 16, 16)] = lax.bitwise_and(pv, 0x3FFF)
                dst_v[j, pl.ds(k * 16, 16)] = lax.shift_right_logical(pv, 14)
            return carry

        lax.fori_loop(0, _HCH, unpack, 0)

        pltpu.async_copy(x_hbm.at[src_v.at[0]], rows0, gsem0)
        pltpu.async_copy(x_hbm.at[src_v.at[1]], rows1, gsem1)

        def step(g, carry):
            for b, (rows, gsem) in enumerate(((rows0, gsem0),
                                              (rows1, gsem1))):
                j = g * 2 + b
                pltpu.make_async_copy(x_hbm.at[src_v.at[j]], rows,
                                      gsem).wait()
                pltpu.sync_copy(rows, acc.at[dst_v.at[j]], add=True)
                nj = j + 2

                @pl.when(nj < _HCH)
                def _():
                    pltpu.async_copy(x_hbm.at[src_v.at[nj]], rows, gsem)
            return carry

        lax.fori_loop(0, _HCH // 2, step, 0)

    plsc.subcore_barrier()

    @pl.when(s < _NSUB - 1)
    def _():
        pltpu.sync_copy(acc.at[pl.ds(off, _ROWS_A)],
                        out_hbm.at[c, pl.ds(off, _ROWS_A)])

    @pl.when(s == _NSUB - 1)
    def _():
        pltpu.sync_copy(acc.at[pl.ds(15 * _ROWS_A, _ROWS_B)],
                        out_hbm.at[c, pl.ds(15 * _ROWS_A, _ROWS_B)])


@functools.lru_cache(maxsize=None)
def _get_sc_agg():
    return pl.kernel(
        _sc_agg_body,
        out_type=jax.ShapeDtypeStruct((_NCORE, _N, _D), jnp.float32),
        mesh=plsc.VectorSubcoreMesh(core_axis_name="c", subcore_axis_name="s"),
        scratch_types=[
            pltpu.VMEM((_HCH, _K), jnp.int32),
            pltpu.VMEM((_HCH, _K), jnp.int32),
            pltpu.VMEM((_HCH, _K), jnp.int32),
            pltpu.VMEM((_K, _D), jnp.float32),
            pltpu.VMEM((_K, _D), jnp.float32),
            pltpu.VMEM_SHARED((_NPAD, _D), jnp.float32),
            pltpu.SemaphoreType.DMA,
            pltpu.SemaphoreType.DMA,
        ],
    )


def _sc_agg(h, idx3):
    return _get_sc_agg()(h, idx3)


def _k1_body(x_ref, p_ref, w1_ref, b1_ref, w2_ref, b2_ref, h2_ref, st_ref):
    i = pl.program_id(0)
    t = p_ref[0] + p_ref[1] - x_ref[...]
    h1 = jnp.maximum(
        jnp.dot(t, w1_ref[...], preferred_element_type=jnp.float32)
        + b1_ref[...], 0.0)
    h2 = jnp.maximum(
        jnp.dot(h1, w2_ref[...], preferred_element_type=jnp.float32)
        + b2_ref[...], 0.0)
    h2_ref[...] = h2

    @pl.when(i == 0)
    def _():
        st_ref[...] = jnp.zeros_like(st_ref)

    upd = jnp.concatenate(
        [jnp.sum(h2, axis=0, keepdims=True),
         jnp.sum(h2 * h2, axis=0, keepdims=True)], axis=0)
    st_ref[0:2, :] = st_ref[0:2, :] + upd


def _k1(x, p, w1, b1, w2, b2):
    return pl.pallas_call(
        _k1_body,
        grid=(_NBLK,),
        in_specs=[
            pl.BlockSpec((_BLK, _D), lambda i: (i, 0)),
            pl.BlockSpec((2, _BLK, _D), lambda i: (0, i, 0)),
            pl.BlockSpec((_D, _D), lambda i: (0, 0)),
            pl.BlockSpec((1, _D), lambda i: (0, 0)),
            pl.BlockSpec((_D, _D), lambda i: (0, 0)),
            pl.BlockSpec((1, _D), lambda i: (0, 0)),
        ],
        out_specs=[
            pl.BlockSpec((_BLK, _D), lambda i: (i, 0)),
            pl.BlockSpec((8, _D), lambda i: (0, 0)),
        ],
        out_shape=[
            jax.ShapeDtypeStruct((_N, _D), jnp.float32),
            jax.ShapeDtypeStruct((8, _D), jnp.float32),
        ],
    )(x, p, w1, b1.reshape(1, _D), w2, b2.reshape(1, _D))


def _norm(h2, st_ref, g_ref, be_ref):
    mean = st_ref[0:1, :] * (1.0 / _N)
    var = st_ref[1:2, :] * (1.0 / _N) - mean * mean
    rstd = lax.rsqrt(var + 1e-5)
    return (h2 - mean) * (rstd * g_ref[...]) + be_ref[...]


def _k2_body(h2_ref, st_ref, g_ref, be_ref, out_ref):
    out_ref[...] = _norm(h2_ref[...], st_ref, g_ref, be_ref)


def _k2(h2, st, g, be):
    return pl.pallas_call(
        _k2_body,
        grid=(_NBLK,),
        in_specs=[
            pl.BlockSpec((_BLK, _D), lambda i: (i, 0)),
            pl.BlockSpec((8, _D), lambda i: (0, 0)),
            pl.BlockSpec((1, _D), lambda i: (0, 0)),
            pl.BlockSpec((1, _D), lambda i: (0, 0)),
        ],
        out_specs=pl.BlockSpec((_BLK, _D), lambda i: (i, 0)),
        out_shape=jax.ShapeDtypeStruct((_N, _D), jnp.float32),
    )(h2, st, g.reshape(1, _D), be.reshape(1, _D))


def _k3_body(h2_ref, st_ref, g_ref, be_ref, b_ref, fw_ref, fb_ref, sw_ref,
             sb_ref, out_ref, sums, cnt):
    i = pl.program_id(0)
    h = _norm(h2_ref[...], st_ref, g_ref, be_ref)
    lbl = b_ref[0]
    ids = lax.broadcasted_iota(jnp.int32, (_G, _BLK), 0)
    oh = (lbl == ids).astype(jnp.float32)

    @pl.when(i == 0)
    def _():
        sums[...] = jnp.zeros_like(sums)
        cnt[...] = jnp.zeros_like(cnt)

    sums[...] = sums[...] + jnp.dot(oh, h, preferred_element_type=jnp.float32)
    cnt[...] = cnt[...] + jnp.broadcast_to(
        jnp.sum(oh, axis=1, keepdims=True), (_G, _D))

    @pl.when(i == _NBLK - 1)
    def _():
        pooled = sums[...] / jnp.maximum(cnt[...], 1.0)
        hh = jnp.maximum(
            jnp.dot(pooled, fw_ref[...], preferred_element_type=jnp.float32)
            + fb_ref[...], 0.0)
        logits = jnp.dot(hh, sw_ref[...],
                         preferred_element_type=jnp.float32) + sb_ref[...]
        m = jnp.max(logits, axis=-1, keepdims=True)
        lse = jnp.log(jnp.sum(jnp.exp(logits - m), axis=-1,
                              keepdims=True)) + m
        out_ref[...] = logits - lse


def _k3(h2, st, g, be, batch3, fw, fb, sw, sb):
    return pl.pallas_call(
        _k3_body,
        grid=(_NBLK,),
        in_specs=[
            pl.BlockSpec((_BLK, _D), lambda i: (i, 0)),
            pl.BlockSpec((8, _D), lambda i: (0, 0)),
            pl.BlockSpec((1, _D), lambda i: (0, 0)),
            pl.BlockSpec((1, _D), lambda i: (0, 0)),
            pl.BlockSpec((1, 1, _BLK), lambda i: (i, 0, 0)),
            pl.BlockSpec((_D, _D), lambda i: (0, 0)),
            pl.BlockSpec((1, _D), lambda i: (0, 0)),
            pl.BlockSpec((_D, 16), lambda i: (0, 0)),
            pl.BlockSpec((1, 16), lambda i: (0, 0)),
        ],
        out_specs=pl.BlockSpec((_G, 16), lambda i: (0, 0)),
        out_shape=jax.ShapeDtypeStruct((_G, 16), jnp.float32),
        scratch_shapes=[
            pltpu.VMEM((_G, _D), jnp.float32),
            pltpu.VMEM((_G, _D), jnp.float32),
        ],
    )(h2, st, g.reshape(1, _D), be.reshape(1, _D), batch3,
      fw, fb.reshape(1, _D), sw, sb.reshape(1, 16))


def kernel(x, edge_index, batch,
           W1_0, b1_0, W2_0, b2_0, g_0, be_0,
           W1_1, b1_1, W2_1, b2_1, g_1, be_1,
           W1_2, b1_2, W2_2, b2_2, g_2, be_2,
           first_W, first_b, second_W, second_b):
    padlen = _EPAD - _E
    fill = jnp.arange(padlen, dtype=jnp.int32)
    src3 = jnp.concatenate(
        [edge_index[0], (fill * 97) % _N]).reshape(_NW, _NCH, _K)
    dst3 = jnp.concatenate(
        [edge_index[1], _N + (fill % 16)]).reshape(_NW, _NCH, _K)
    idx3 = jnp.pad(((dst3 << 14) | src3).reshape(_NW * _NCH, _K),
                   ((0, 16384 - _NW * _NCH), (0, 0)))
    batch3 = batch.reshape(_NBLK, 1, _BLK)

    h = x
    params = ((W1_0, b1_0, W2_0, b2_0, g_0, be_0),
              (W1_1, b1_1, W2_1, b2_1, g_1, be_1),
              (W1_2, b1_2, W2_2, b2_2, g_2, be_2))
    for li, (w1, b1, w2, b2, g, be) in enumerate(params):
        p = _sc_agg(h, idx3)
        h2, st = _k1(h, p, w1, b1, w2, b2)
        if li < 2:
            h = _k2(h2, st, g, be)
        else:
            out = _k3(h2, st, g, be, batch3,
                      first_W, first_b, second_W, second_b)
    return out


# speedup vs baseline: 1.0279x; 1.0200x over previous
import functools

import jax
import jax.numpy as jnp
from jax import lax
from jax.experimental import pallas as pl
from jax.experimental.pallas import tpu as pltpu
from jax.experimental.pallas import tpu_sc as plsc

_N = 10000
_E = 320000
_D = 128
_G = 64
_NSUB = 16
_NCORE = 2
_NW = _NSUB * _NCORE
_K = 128
_NCH = 80
_HCH = 40
_EPAD = _NW * _NCH * _K
_ROWS_A = 640
_ROWS_B = _N - 15 * _ROWS_A
_NPAD = _N + 16
_BLK = 2000
_NBLK = _N // _BLK


def _sc_agg_body(x_hbm, idx_hbm, out_hbm,
                 pk_v, src_v, dst_v, rows0, rows1, acc, gsem0, gsem1, isem):
    c = lax.axis_index("c")
    s = lax.axis_index("s")
    wid = c * _NSUB + s

    off = pl.multiple_of(s * _ROWS_A, 8)

    @pl.when(s < _NSUB - 1)
    def _():
        pltpu.async_copy(x_hbm.at[pl.ds(off, _ROWS_A)],
                         acc.at[pl.ds(off, _ROWS_A)], isem)

    @pl.when(s == _NSUB - 1)
    def _():
        pltpu.async_copy(x_hbm.at[pl.ds(15 * _ROWS_A, _ROWS_B)],
                         acc.at[pl.ds(15 * _ROWS_A, _ROWS_B)], isem)

    for half in range(2):
        pltpu.sync_copy(
            idx_hbm.at[pl.ds(pl.multiple_of(wid * _NCH + half * _HCH, 8),
                             _HCH)], pk_v)

        def unpack(j, carry):
            for k in range(_K // 16):
                pv = pk_v[j, pl.ds(k * 16, 16)]
                src_v[j, pl.ds(k * 16, 16)] = lax.bitwise_and(pv, 0x3FFF)
                dst_v[j, pl.ds(k * 16, 16)] = lax.shift_right_logical(pv, 14)
            return carry

        unpack(0, 0)
        unpack(1, 0)
        pltpu.async_copy(x_hbm.at[src_v.at[0]], rows0, gsem0)
        pltpu.async_copy(x_hbm.at[src_v.at[1]], rows1, gsem1)
        lax.fori_loop(2, _HCH, unpack, 0)

        if half == 0:
            @pl.when(s < _NSUB - 1)
            def _():
                pltpu.make_async_copy(
                    x_hbm.at[pl.ds(off, _ROWS_A)],
                    acc.at[pl.ds(off, _ROWS_A)], isem).wait()

            @pl.when(s == _NSUB - 1)
            def _():
                pltpu.make_async_copy(
                    x_hbm.at[pl.ds(15 * _ROWS_A, _ROWS_B)],
                    acc.at[pl.ds(15 * _ROWS_A, _ROWS_B)], isem).wait()

            plsc.subcore_barrier()

        def step(g, carry):
            for b, (rows, gsem) in enumerate(((rows0, gsem0),
                                              (rows1, gsem1))):
                j = g * 2 + b
                pltpu.make_async_copy(x_hbm.at[src_v.at[j]], rows,
                                      gsem).wait()
                pltpu.sync_copy(rows, acc.at[dst_v.at[j]], add=True)
                nj = j + 2

                @pl.when(nj < _HCH)
                def _():
                    pltpu.async_copy(x_hbm.at[src_v.at[nj]], rows, gsem)
            return carry

        lax.fori_loop(0, _HCH // 2, step, 0)

    plsc.subcore_barrier()

    @pl.when(s < _NSUB - 1)
    def _():
        pltpu.sync_copy(acc.at[pl.ds(off, _ROWS_A)],
                        out_hbm.at[c, pl.ds(off, _ROWS_A)])

    @pl.when(s == _NSUB - 1)
    def _():
        pltpu.sync_copy(acc.at[pl.ds(15 * _ROWS_A, _ROWS_B)],
                        out_hbm.at[c, pl.ds(15 * _ROWS_A, _ROWS_B)])


@functools.lru_cache(maxsize=None)
def _get_sc_agg():
    return pl.kernel(
        _sc_agg_body,
        out_type=jax.ShapeDtypeStruct((_NCORE, _N, _D), jnp.float32),
        mesh=plsc.VectorSubcoreMesh(core_axis_name="c", subcore_axis_name="s"),
        scratch_types=[
            pltpu.VMEM((_HCH, _K), jnp.int32),
            pltpu.VMEM((_HCH, _K), jnp.int32),
            pltpu.VMEM((_HCH, _K), jnp.int32),
            pltpu.VMEM((_K, _D), jnp.float32),
            pltpu.VMEM((_K, _D), jnp.float32),
            pltpu.VMEM_SHARED((_NPAD, _D), jnp.float32),
            pltpu.SemaphoreType.DMA,
            pltpu.SemaphoreType.DMA,
            pltpu.SemaphoreType.DMA,
        ],
    )


def _sc_agg(h, idx3):
    return _get_sc_agg()(h, idx3)


def _k1_body(x_ref, p_ref, w1_ref, b1_ref, w2_ref, b2_ref, h2_ref, st_ref):
    i = pl.program_id(0)
    t = p_ref[0] + p_ref[1] - x_ref[...]
    h1 = jnp.maximum(
        jnp.dot(t, w1_ref[...], preferred_element_type=jnp.float32)
        + b1_ref[...], 0.0)
    h2 = jnp.maximum(
        jnp.dot(h1, w2_ref[...], preferred_element_type=jnp.float32)
        + b2_ref[...], 0.0)
    h2_ref[...] = h2

    @pl.when(i == 0)
    def _():
        st_ref[...] = jnp.zeros_like(st_ref)

    upd = jnp.concatenate(
        [jnp.sum(h2, axis=0, keepdims=True),
         jnp.sum(h2 * h2, axis=0, keepdims=True)], axis=0)
    st_ref[0:2, :] = st_ref[0:2, :] + upd


def _k1(x, p, w1, b1, w2, b2):
    return pl.pallas_call(
        _k1_body,
        grid=(_NBLK,),
        in_specs=[
            pl.BlockSpec((_BLK, _D), lambda i: (i, 0)),
            pl.BlockSpec((2, _BLK, _D), lambda i: (0, i, 0)),
            pl.BlockSpec((_D, _D), lambda i: (0, 0)),
            pl.BlockSpec((1, _D), lambda i: (0, 0)),
            pl.BlockSpec((_D, _D), lambda i: (0, 0)),
            pl.BlockSpec((1, _D), lambda i: (0, 0)),
        ],
        out_specs=[
            pl.BlockSpec((_BLK, _D), lambda i: (i, 0)),
            pl.BlockSpec((8, _D), lambda i: (0, 0)),
        ],
        out_shape=[
            jax.ShapeDtypeStruct((_N, _D), jnp.float32),
            jax.ShapeDtypeStruct((8, _D), jnp.float32),
        ],
    )(x, p, w1, b1.reshape(1, _D), w2, b2.reshape(1, _D))


def _norm(h2, st_ref, g_ref, be_ref):
    mean = st_ref[0:1, :] * (1.0 / _N)
    var = st_ref[1:2, :] * (1.0 / _N) - mean * mean
    rstd = lax.rsqrt(var + 1e-5)
    return (h2 - mean) * (rstd * g_ref[...]) + be_ref[...]


def _k2_body(h2_ref, st_ref, g_ref, be_ref, out_ref):
    out_ref[...] = _norm(h2_ref[...], st_ref, g_ref, be_ref)


def _k2(h2, st, g, be):
    return pl.pallas_call(
        _k2_body,
        grid=(_NBLK,),
        in_specs=[
            pl.BlockSpec((_BLK, _D), lambda i: (i, 0)),
            pl.BlockSpec((8, _D), lambda i: (0, 0)),
            pl.BlockSpec((1, _D), lambda i: (0, 0)),
            pl.BlockSpec((1, _D), lambda i: (0, 0)),
        ],
        out_specs=pl.BlockSpec((_BLK, _D), lambda i: (i, 0)),
        out_shape=jax.ShapeDtypeStruct((_N, _D), jnp.float32),
    )(h2, st, g.reshape(1, _D), be.reshape(1, _D))


def _k3_body(h2_ref, st_ref, g_ref, be_ref, b_ref, fw_ref, fb_ref, sw_ref,
             sb_ref, out_ref, sums, cnt):
    i = pl.program_id(0)
    h = _norm(h2_ref[...], st_ref, g_ref, be_ref)
    lbl = b_ref[0]
    ids = lax.broadcasted_iota(jnp.int32, (_G, _BLK), 0)
    oh = (lbl == ids).astype(jnp.float32)

    @pl.when(i == 0)
    def _():
        sums[...] = jnp.zeros_like(sums)
        cnt[...] = jnp.zeros_like(cnt)

    sums[...] = sums[...] + jnp.dot(oh, h, preferred_element_type=jnp.float32)
    cnt[...] = cnt[...] + jnp.broadcast_to(
        jnp.sum(oh, axis=1, keepdims=True), (_G, _D))

    @pl.when(i == _NBLK - 1)
    def _():
        pooled = sums[...] / jnp.maximum(cnt[...], 1.0)
        hh = jnp.maximum(
            jnp.dot(pooled, fw_ref[...], preferred_element_type=jnp.float32)
            + fb_ref[...], 0.0)
        logits = jnp.dot(hh, sw_ref[...],
                         preferred_element_type=jnp.float32) + sb_ref[...]
        m = jnp.max(logits, axis=-1, keepdims=True)
        lse = jnp.log(jnp.sum(jnp.exp(logits - m), axis=-1,
                              keepdims=True)) + m
        out_ref[...] = logits - lse


def _k3(h2, st, g, be, batch3, fw, fb, sw, sb):
    return pl.pallas_call(
        _k3_body,
        grid=(_NBLK,),
        in_specs=[
            pl.BlockSpec((_BLK, _D), lambda i: (i, 0)),
            pl.BlockSpec((8, _D), lambda i: (0, 0)),
            pl.BlockSpec((1, _D), lambda i: (0, 0)),
            pl.BlockSpec((1, _D), lambda i: (0, 0)),
            pl.BlockSpec((1, 1, _BLK), lambda i: (i, 0, 0)),
            pl.BlockSpec((_D, _D), lambda i: (0, 0)),
            pl.BlockSpec((1, _D), lambda i: (0, 0)),
            pl.BlockSpec((_D, 16), lambda i: (0, 0)),
            pl.BlockSpec((1, 16), lambda i: (0, 0)),
        ],
        out_specs=pl.BlockSpec((_G, 16), lambda i: (0, 0)),
        out_shape=jax.ShapeDtypeStruct((_G, 16), jnp.float32),
        scratch_shapes=[
            pltpu.VMEM((_G, _D), jnp.float32),
            pltpu.VMEM((_G, _D), jnp.float32),
        ],
    )(h2, st, g.reshape(1, _D), be.reshape(1, _D), batch3,
      fw, fb.reshape(1, _D), sw, sb.reshape(1, 16))


def kernel(x, edge_index, batch,
           W1_0, b1_0, W2_0, b2_0, g_0, be_0,
           W1_1, b1_1, W2_1, b2_1, g_1, be_1,
           W1_2, b1_2, W2_2, b2_2, g_2, be_2,
           first_W, first_b, second_W, second_b):
    padlen = _EPAD - _E
    fill = jnp.arange(padlen, dtype=jnp.int32)
    src3 = jnp.concatenate(
        [edge_index[0], (fill * 97) % _N]).reshape(_NW, _NCH, _K)
    dst3 = jnp.concatenate(
        [edge_index[1], _N + (fill % 16)]).reshape(_NW, _NCH, _K)
    idx3 = jnp.pad(((dst3 << 14) | src3).reshape(_NW * _NCH, _K),
                   ((0, 16384 - _NW * _NCH), (0, 0)))
    batch3 = batch.reshape(_NBLK, 1, _BLK)

    h = x
    params = ((W1_0, b1_0, W2_0, b2_0, g_0, be_0),
              (W1_1, b1_1, W2_1, b2_1, g_1, be_1),
              (W1_2, b1_2, W2_2, b2_2, g_2, be_2))
    for li, (w1, b1, w2, b2, g, be) in enumerate(params):
        p = _sc_agg(h, idx3)
        h2, st = _k1(h, p, w1, b1, w2, b2)
        if li < 2:
            h = _k2(h2, st, g, be)
        else:
            out = _k3(h2, st, g, be, batch3,
                      first_W, first_b, second_W, second_b)
    return out


# speedup vs baseline: 1.0542x; 1.0256x over previous
import functools

import jax
import jax.numpy as jnp
from jax import lax
from jax.experimental import pallas as pl
from jax.experimental.pallas import tpu as pltpu
from jax.experimental.pallas import tpu_sc as plsc

_N = 10000
_E = 320000
_D = 128
_G = 64
_NSUB = 16
_NCORE = 2
_NW = _NSUB * _NCORE
_K = 128
_NCH = 80
_HCH = 40
_EPAD = _NW * _NCH * _K
_ROWS_A = 640
_ROWS_B = _N - 15 * _ROWS_A
_NPAD = _N + 16
_BLK = 5000
_NBLK = _N // _BLK


def _sc_agg_body(x_hbm, idx_hbm, out_hbm,
                 pk_v, src_v, dst_v, rows0, rows1, acc, gsem0, gsem1, isem):
    c = lax.axis_index("c")
    s = lax.axis_index("s")
    wid = c * _NSUB + s

    off = pl.multiple_of(s * _ROWS_A, 8)

    @pl.when(s < _NSUB - 1)
    def _():
        pltpu.async_copy(x_hbm.at[pl.ds(off, _ROWS_A)],
                         acc.at[pl.ds(off, _ROWS_A)], isem)

    @pl.when(s == _NSUB - 1)
    def _():
        pltpu.async_copy(x_hbm.at[pl.ds(15 * _ROWS_A, _ROWS_B)],
                         acc.at[pl.ds(15 * _ROWS_A, _ROWS_B)], isem)

    for half in range(2):
        pltpu.sync_copy(
            idx_hbm.at[pl.ds(pl.multiple_of(wid * _NCH + half * _HCH, 8),
                             _HCH)], pk_v)

        def unpack(j, carry):
            for k in range(_K // 16):
                pv = pk_v[j, pl.ds(k * 16, 16)]
                src_v[j, pl.ds(k * 16, 16)] = lax.bitwise_and(pv, 0x3FFF)
                dst_v[j, pl.ds(k * 16, 16)] = lax.shift_right_logical(pv, 14)
            return carry

        unpack(0, 0)
        unpack(1, 0)
        pltpu.async_copy(x_hbm.at[src_v.at[0]], rows0, gsem0)
        pltpu.async_copy(x_hbm.at[src_v.at[1]], rows1, gsem1)
        lax.fori_loop(2, _HCH, unpack, 0)

        if half == 0:
            @pl.when(s < _NSUB - 1)
            def _():
                pltpu.make_async_copy(
                    x_hbm.at[pl.ds(off, _ROWS_A)],
                    acc.at[pl.ds(off, _ROWS_A)], isem).wait()

            @pl.when(s == _NSUB - 1)
            def _():
                pltpu.make_async_copy(
                    x_hbm.at[pl.ds(15 * _ROWS_A, _ROWS_B)],
                    acc.at[pl.ds(15 * _ROWS_A, _ROWS_B)], isem).wait()

            plsc.subcore_barrier()

        def step(g, carry):
            for b, (rows, gsem) in enumerate(((rows0, gsem0),
                                              (rows1, gsem1))):
                j = g * 2 + b
                pltpu.make_async_copy(x_hbm.at[src_v.at[j]], rows,
                                      gsem).wait()
                pltpu.sync_copy(rows, acc.at[dst_v.at[j]], add=True)
                nj = j + 2

                @pl.when(nj < _HCH)
                def _():
                    pltpu.async_copy(x_hbm.at[src_v.at[nj]], rows, gsem)
            return carry

        lax.fori_loop(0, _HCH // 2, step, 0)

    plsc.subcore_barrier()

    @pl.when(s < _NSUB - 1)
    def _():
        pltpu.sync_copy(acc.at[pl.ds(off, _ROWS_A)],
                        out_hbm.at[c, pl.ds(off, _ROWS_A)])

    @pl.when(s == _NSUB - 1)
    def _():
        pltpu.sync_copy(acc.at[pl.ds(15 * _ROWS_A, _ROWS_B)],
                        out_hbm.at[c, pl.ds(15 * _ROWS_A, _ROWS_B)])


@functools.lru_cache(maxsize=None)
def _get_sc_agg():
    return pl.kernel(
        _sc_agg_body,
        out_type=jax.ShapeDtypeStruct((_NCORE, _N, _D), jnp.float32),
        mesh=plsc.VectorSubcoreMesh(core_axis_name="c", subcore_axis_name="s"),
        scratch_types=[
            pltpu.VMEM((_HCH, _K), jnp.int32),
            pltpu.VMEM((_HCH, _K), jnp.int32),
            pltpu.VMEM((_HCH, _K), jnp.int32),
            pltpu.VMEM((_K, _D), jnp.float32),
            pltpu.VMEM((_K, _D), jnp.float32),
            pltpu.VMEM_SHARED((_NPAD, _D), jnp.float32),
            pltpu.SemaphoreType.DMA,
            pltpu.SemaphoreType.DMA,
            pltpu.SemaphoreType.DMA,
        ],
    )


def _sc_agg(h, idx3):
    return _get_sc_agg()(h, idx3)


def _k1_body(x_ref, p_ref, w1_ref, b1_ref, w2_ref, b2_ref, h2_ref, st_ref):
    i = pl.program_id(0)
    t = p_ref[0] + p_ref[1] - x_ref[...]
    h1 = jnp.maximum(
        jnp.dot(t, w1_ref[...], preferred_element_type=jnp.float32)
        + b1_ref[...], 0.0)
    h2 = jnp.maximum(
        jnp.dot(h1, w2_ref[...], preferred_element_type=jnp.float32)
        + b2_ref[...], 0.0)
    h2_ref[...] = h2

    @pl.when(i == 0)
    def _():
        st_ref[...] = jnp.zeros_like(st_ref)

    upd = jnp.concatenate(
        [jnp.sum(h2, axis=0, keepdims=True),
         jnp.sum(h2 * h2, axis=0, keepdims=True)], axis=0)
    st_ref[0:2, :] = st_ref[0:2, :] + upd


def _k1(x, p, w1, b1, w2, b2):
    return pl.pallas_call(
        _k1_body,
        grid=(_NBLK,),
        in_specs=[
            pl.BlockSpec((_BLK, _D), lambda i: (i, 0)),
            pl.BlockSpec((2, _BLK, _D), lambda i: (0, i, 0)),
            pl.BlockSpec((_D, _D), lambda i: (0, 0)),
            pl.BlockSpec((1, _D), lambda i: (0, 0)),
            pl.BlockSpec((_D, _D), lambda i: (0, 0)),
            pl.BlockSpec((1, _D), lambda i: (0, 0)),
        ],
        out_specs=[
            pl.BlockSpec((_BLK, _D), lambda i: (i, 0)),
            pl.BlockSpec((8, _D), lambda i: (0, 0)),
        ],
        out_shape=[
            jax.ShapeDtypeStruct((_N, _D), jnp.float32),
            jax.ShapeDtypeStruct((8, _D), jnp.float32),
        ],
    )(x, p, w1, b1.reshape(1, _D), w2, b2.reshape(1, _D))


def _norm(h2, st_ref, g_ref, be_ref):
    mean = st_ref[0:1, :] * (1.0 / _N)
    var = st_ref[1:2, :] * (1.0 / _N) - mean * mean
    rstd = lax.rsqrt(var + 1e-5)
    return (h2 - mean) * (rstd * g_ref[...]) + be_ref[...]


def _k2_body(h2_ref, st_ref, g_ref, be_ref, out_ref):
    out_ref[...] = _norm(h2_ref[...], st_ref, g_ref, be_ref)


def _k2(h2, st, g, be):
    return pl.pallas_call(
        _k2_body,
        grid=(_NBLK,),
        in_specs=[
            pl.BlockSpec((_BLK, _D), lambda i: (i, 0)),
            pl.BlockSpec((8, _D), lambda i: (0, 0)),
            pl.BlockSpec((1, _D), lambda i: (0, 0)),
            pl.BlockSpec((1, _D), lambda i: (0, 0)),
        ],
        out_specs=pl.BlockSpec((_BLK, _D), lambda i: (i, 0)),
        out_shape=jax.ShapeDtypeStruct((_N, _D), jnp.float32),
    )(h2, st, g.reshape(1, _D), be.reshape(1, _D))


def _k3_body(h2_ref, st_ref, g_ref, be_ref, b_ref, fw_ref, fb_ref, sw_ref,
             sb_ref, out_ref, sums, cnt):
    i = pl.program_id(0)
    h = _norm(h2_ref[...], st_ref, g_ref, be_ref)
    lbl = b_ref[0]
    ids = lax.broadcasted_iota(jnp.int32, (_G, _BLK), 0)
    oh = (lbl == ids).astype(jnp.float32)

    @pl.when(i == 0)
    def _():
        sums[...] = jnp.zeros_like(sums)
        cnt[...] = jnp.zeros_like(cnt)

    sums[...] = sums[...] + jnp.dot(oh, h, preferred_element_type=jnp.float32)
    cnt[...] = cnt[...] + jnp.broadcast_to(
        jnp.sum(oh, axis=1, keepdims=True), (_G, _D))

    @pl.when(i == _NBLK - 1)
    def _():
        pooled = sums[...] / jnp.maximum(cnt[...], 1.0)
        hh = jnp.maximum(
            jnp.dot(pooled, fw_ref[...], preferred_element_type=jnp.float32)
            + fb_ref[...], 0.0)
        logits = jnp.dot(hh, sw_ref[...],
                         preferred_element_type=jnp.float32) + sb_ref[...]
        m = jnp.max(logits, axis=-1, keepdims=True)
        lse = jnp.log(jnp.sum(jnp.exp(logits - m), axis=-1,
                              keepdims=True)) + m
        out_ref[...] = logits - lse


def _k3(h2, st, g, be, batch3, fw, fb, sw, sb):
    return pl.pallas_call(
        _k3_body,
        grid=(_NBLK,),
        in_specs=[
            pl.BlockSpec((_BLK, _D), lambda i: (i, 0)),
            pl.BlockSpec((8, _D), lambda i: (0, 0)),
            pl.BlockSpec((1, _D), lambda i: (0, 0)),
            pl.BlockSpec((1, _D), lambda i: (0, 0)),
            pl.BlockSpec((1, 1, _BLK), lambda i: (i, 0, 0)),
            pl.BlockSpec((_D, _D), lambda i: (0, 0)),
            pl.BlockSpec((1, _D), lambda i: (0, 0)),
            pl.BlockSpec((_D, 16), lambda i: (0, 0)),
            pl.BlockSpec((1, 16), lambda i: (0, 0)),
        ],
        out_specs=pl.BlockSpec((_G, 16), lambda i: (0, 0)),
        out_shape=jax.ShapeDtypeStruct((_G, 16), jnp.float32),
        scratch_shapes=[
            pltpu.VMEM((_G, _D), jnp.float32),
            pltpu.VMEM((_G, _D), jnp.float32),
        ],
    )(h2, st, g.reshape(1, _D), be.reshape(1, _D), batch3,
      fw, fb.reshape(1, _D), sw, sb.reshape(1, 16))


def kernel(x, edge_index, batch,
           W1_0, b1_0, W2_0, b2_0, g_0, be_0,
           W1_1, b1_1, W2_1, b2_1, g_1, be_1,
           W1_2, b1_2, W2_2, b2_2, g_2, be_2,
           first_W, first_b, second_W, second_b):
    padlen = _EPAD - _E
    fill = jnp.arange(padlen, dtype=jnp.int32)
    src3 = jnp.concatenate(
        [edge_index[0], (fill * 97) % _N]).reshape(_NW, _NCH, _K)
    dst3 = jnp.concatenate(
        [edge_index[1], _N + (fill % 16)]).reshape(_NW, _NCH, _K)
    idx3 = jnp.pad(((dst3 << 14) | src3).reshape(_NW * _NCH, _K),
                   ((0, 16384 - _NW * _NCH), (0, 0)))
    batch3 = batch.reshape(_NBLK, 1, _BLK)

    h = x
    params = ((W1_0, b1_0, W2_0, b2_0, g_0, be_0),
              (W1_1, b1_1, W2_1, b2_1, g_1, be_1),
              (W1_2, b1_2, W2_2, b2_2, g_2, be_2))
    for li, (w1, b1, w2, b2, g, be) in enumerate(params):
        p = _sc_agg(h, idx3)
        h2, st = _k1(h, p, w1, b1, w2, b2)
        if li < 2:
            h = _k2(h2, st, g, be)
        else:
            out = _k3(h2, st, g, be, batch3,
                      first_W, first_b, second_W, second_b)
    return out
